# baseline (speedup 1.0000x reference)
import jax
import jax.numpy as jnp
from jax.experimental import pallas as pl
from jax.experimental.pallas import tpu as pltpu

IN_F = 10
OUT_F = 5
GROUP = 128
BR = 512


def _round_up(n: int, m: int) -> int:
    return ((n + m - 1) // m) * m


def _packed_linear_kernel(x_ref, w_ref, b_ref, o_ref):
    x = x_ref[...]
    xr = x.reshape(x.shape[0] // IN_F, IN_F * 128)
    acc = jnp.dot(xr, w_ref[...], preferred_element_type=jnp.float32)
    acc = acc + b_ref[...]
    o_ref[...] = acc.reshape(acc.shape[0] * OUT_F, 128).astype(o_ref.dtype)


@jax.jit
def _forward(x, w_packed, b_packed):
    B, in_f = x.shape
    assert in_f == IN_F

    w = w_packed[:, :OUT_F]
    b = b_packed[:, :OUT_F]
    w_big = jnp.kron(jnp.eye(GROUP, dtype=x.dtype), w)
    b_big = jnp.tile(b, (1, GROUP))

    b_pad = _round_up(B, GROUP * BR)
    xp = jnp.pad(x, ((0, b_pad - B), (0, 0))) if b_pad != B else x
    x2 = xp.reshape(b_pad * IN_F // 128, 128)
    grid = b_pad // (GROUP * BR)

    y2 = pl.pallas_call(
        _packed_linear_kernel,
        out_shape=jax.ShapeDtypeStruct((b_pad * OUT_F // 128, 128), x.dtype),
        grid=(grid,),
        in_specs=[
            pl.BlockSpec((IN_F * BR, 128), lambda i: (i, 0)),
            pl.BlockSpec((IN_F * GROUP, OUT_F * GROUP), lambda i: (0, 0)),
            pl.BlockSpec((1, OUT_F * GROUP), lambda i: (0, 0)),
        ],
        out_specs=pl.BlockSpec((OUT_F * BR, 128), lambda i: (i, 0)),
        compiler_params=pltpu.CompilerParams(
            dimension_semantics=("parallel",),
        ),
    )(x2, w_big, b_big)

    y = y2.reshape(b_pad, OUT_F)
    return y[:B] if b_pad != B else y


def kernel(x, w_packed, b_packed):
    return _forward(x, w_packed, b_packed)


# speedup vs baseline: 34.3193x; 34.3193x over previous
import jax
import jax.numpy as jnp
from jax.experimental import pallas as pl
from jax.experimental.pallas import tpu as pltpu

IN_F = 10
OUT_F = 5
CB = 65536


def _round_up(n: int, m: int) -> int:
    return ((n + m - 1) // m) * m


def _linear_t_kernel(xt_ref, w_ref, b_ref, o_ref):
    acc = jnp.dot(w_ref[...], xt_ref[...], preferred_element_type=jnp.float32)
    o_ref[...] = (acc + b_ref[...]).astype(o_ref.dtype)


@jax.jit
def _forward(x, w_packed, b_packed):
    B, in_f = x.shape
    assert in_f == IN_F

    w = w_packed[:, :OUT_F].T
    b = b_packed[:, :OUT_F].T

    b_pad = _round_up(B, CB)
    xp = jnp.pad(x, ((0, b_pad - B), (0, 0))) if b_pad != B else x
    xt = xp.T

    yt = pl.pallas_call(
        _linear_t_kernel,
        out_shape=jax.ShapeDtypeStruct((OUT_F, b_pad), x.dtype),
        grid=(b_pad // CB,),
        in_specs=[
            pl.BlockSpec((IN_F, CB), lambda i: (0, i)),
            pl.BlockSpec((OUT_F, IN_F), lambda i: (0, 0)),
            pl.BlockSpec((OUT_F, 1), lambda i: (0, 0)),
        ],
        out_specs=pl.BlockSpec((OUT_F, CB), lambda i: (0, i)),
        compiler_params=pltpu.CompilerParams(
            dimension_semantics=("parallel",),
        ),
    )(xt, w, b)

    y = yt.T
    return y[:B] if b_pad != B else y


def kernel(x, w_packed, b_packed):
    return _forward(x, w_packed, b_packed)
